# baseline (speedup 1.0000x reference)
import functools

import jax
import jax.numpy as jnp
from jax import lax
from jax.experimental import pallas as pl
from jax.experimental.pallas import tpu as pltpu
from jax.experimental.pallas import tpu_sc as plsc

NN = 10000
NPAD = 10240
EE = 160000
EPAD = 163840
TW = 80
CH = 128
NWORK = 32
CPW = EPAD // (NWORK * CH)
HID = 64
BE = 2048
BN = 1024
F32 = jnp.float32


@functools.lru_cache(maxsize=None)
def _gather_kernel():
    mesh = plsc.VectorSubcoreMesh(core_axis_name="c", subcore_axis_name="s")

    @functools.partial(
        pl.kernel, mesh=mesh,
        compiler_params=pltpu.CompilerParams(use_tc_tiling_on_sc=False),
        out_type=(jax.ShapeDtypeStruct((EPAD, TW), F32),
                  jax.ShapeDtypeStruct((EPAD, TW), F32)),
        scratch_types=[pltpu.VMEM((CPW, CH), jnp.int32),
                       pltpu.VMEM((CPW, CH), jnp.int32),
                       pltpu.VMEM((2, CH, TW), F32),
                       pltpu.VMEM((2, CH, TW), F32),
                       [pltpu.SemaphoreType.DMA] * 2,
                       [pltpu.SemaphoreType.DMA] * 2,
                       [pltpu.SemaphoreType.DMA] * 2,
                       [pltpu.SemaphoreType.DMA] * 2])
    def k(t_hbm, row_hbm, col_hbm, orow_hbm, ocol_hbm,
          idxr_v, idxc_v, rbuf_v, cbuf_v, sgr, sgc, sor, soc):
        wid = lax.axis_index("s") * 2 + lax.axis_index("c")
        base = wid * CPW
        pltpu.sync_copy(row_hbm.at[pl.ds(base, CPW)], idxr_v)
        pltpu.sync_copy(col_hbm.at[pl.ds(base, CPW)], idxc_v)

        def start_g(j, b):
            pltpu.async_copy(t_hbm.at[idxr_v.at[j]], rbuf_v.at[b], sgr[b])
            pltpu.async_copy(t_hbm.at[idxc_v.at[j]], cbuf_v.at[b], sgc[b])

        def wait_g(b):
            dummy = orow_hbm.at[pl.ds(0, CH)]
            pltpu.make_async_copy(dummy, rbuf_v.at[b], sgr[b]).wait()
            pltpu.make_async_copy(dummy, cbuf_v.at[b], sgc[b]).wait()

        for b in range(2):
            start_g(b, b)

        def body(i, carry):
            g = i * 2
            for b in range(2):
                j = g + b
                eb = (base + j) * CH
                wait_g(b)
                pltpu.async_copy(rbuf_v.at[b], orow_hbm.at[pl.ds(eb, CH)],
                                 sor[b])
                pltpu.async_copy(cbuf_v.at[b], ocol_hbm.at[pl.ds(eb, CH)],
                                 soc[b])
                pltpu.make_async_copy(rbuf_v.at[b],
                                      orow_hbm.at[pl.ds(eb, CH)],
                                      sor[b]).wait()
                pltpu.make_async_copy(cbuf_v.at[b],
                                      ocol_hbm.at[pl.ds(eb, CH)],
                                      soc[b]).wait()

                @pl.when(j + 2 < CPW)
                def _():
                    start_g(j + 2, b)

            return carry

        lax.fori_loop(0, CPW // 2, body, 0)

    return k


@functools.lru_cache(maxsize=None)
def _scatter_kernel():
    mesh = plsc.VectorSubcoreMesh(core_axis_name="c", subcore_axis_name="s")
    zchunks = NPAD // CH // 16

    @functools.partial(
        pl.kernel, mesh=mesh,
        compiler_params=pltpu.CompilerParams(use_tc_tiling_on_sc=False),
        out_type=jax.ShapeDtypeStruct((2, NPAD, TW), F32),
        scratch_types=[pltpu.VMEM((CPW, CH), jnp.int32),
                       pltpu.VMEM((CH, TW), F32),
                       pltpu.VMEM_SHARED((NPAD, TW), F32)])
    def k(vals_hbm, row_hbm, zero_hbm, p_hbm, idx_v, buf_v, acc_s):
        cid = lax.axis_index("c")
        sid = lax.axis_index("s")
        wid = sid * 2 + cid
        pltpu.sync_copy(zero_hbm, buf_v)

        def zbody(j, carry):
            pltpu.sync_copy(buf_v, acc_s.at[pl.ds((sid * zchunks + j) * CH, CH)])
            return carry

        lax.fori_loop(0, zchunks, zbody, 0)
        plsc.subcore_barrier()

        base = wid * CPW
        pltpu.sync_copy(row_hbm.at[pl.ds(base, CPW)], idx_v)

        def body(j, carry):
            eb = (base + j) * CH
            pltpu.sync_copy(vals_hbm.at[pl.ds(eb, CH)], buf_v)
            pltpu.sync_copy(buf_v, acc_s.at[idx_v.at[j]], add=True)
            return carry

        lax.fori_loop(0, CPW, body, 0)
        plsc.subcore_barrier()

        def obody(j, carry):
            off = (sid * zchunks + j) * CH
            pltpu.sync_copy(acc_s.at[pl.ds(off, CH)], buf_v)
            pltpu.sync_copy(buf_v, p_hbm.at[cid, pl.ds(off, CH)])
            return carry

        lax.fori_loop(0, zchunks, obody, 0)

    return k


def _sc_gather(t, row2d, col2d):
    return _gather_kernel()(t, row2d, col2d)


def _sc_scatter(vals, row2d, zero128):
    return _scatter_kernel()(vals, row2d, zero128)


def _full(shape):
    nd = len(shape)
    return pl.BlockSpec(shape, lambda i: (0,) * nd)


def _edge_body(tr_ref, tc_ref, ea_ref, w1a_ref, w1b_ref, w1g_ref, w1e_ref,
               b1_ref, w2_ref, b2_ref, g_ref, lb_ref, wc1_ref, bc1_ref,
               wc2_ref, out_ref):
    tr = tr_ref[...]
    tc = tc_ref[...]
    hr = tr[:, :HID]
    hc = tc[:, :HID]
    x0, x1, x2 = tr[:, 64:65], tr[:, 65:66], tr[:, 66:67]
    y0, y1, y2 = tc[:, 64:65], tc[:, 65:66], tc[:, 66:67]
    r0, r1, r2 = x0 - y0, x1 - y1, x2 - y2
    rad = r0 * r0 + r1 * r1 + r2 * r2
    dist = jnp.sqrt(rad)
    dot = x0 * y0 + x1 * y1 + x2 * y2
    inv_r = 1.0 / (dist + 1e-8)
    a0, a1, a2 = r0 * inv_r, r1 * inv_r, r2 * inv_r
    c0 = x1 * y2 - x2 * y1
    c1 = x2 * y0 - x0 * y2
    c2 = x0 * y1 - x1 * y0
    cn = jnp.sqrt(c0 * c0 + c1 * c1 + c2 * c2)
    inv_c = 1.0 / (cn + 1e-8)
    b0, b1v, b2v = c0 * inv_c, c1 * inv_c, c2 * inv_c
    e0 = a1 * b2v - a2 * b1v
    e1 = a2 * b0 - a0 * b2v
    e2 = a0 * b1v - a1 * b0
    na = jnp.sqrt(a0 * a0 + a1 * a1 + a2 * a2)
    nb = jnp.sqrt(b0 * b0 + b1v * b1v + b2v * b2v)
    nc = jnp.sqrt(e0 * e0 + e1 * e1 + e2 * e2)
    thr = 1e-6
    mask = (na < thr) | (nb < thr) | (nc < thr)
    so3 = [(a0, 1.0), (b0, 0.0), (e0, 0.0),
           (a1, 0.0), (b1v, 1.0), (e1, 0.0),
           (a2, 0.0), (b2v, 0.0), (e2, 1.0)]
    geo = [rad, dist, dot] + [jnp.where(mask, ident, v) for v, ident in so3]

    u = (jnp.dot(hr, w1a_ref[...], preferred_element_type=F32)
         + jnp.dot(hc, w1b_ref[...], preferred_element_type=F32)
         + jnp.dot(ea_ref[...], w1e_ref[...], preferred_element_type=F32)
         + b1_ref[...])
    w1g = w1g_ref[...]
    for kk, gcol in enumerate(geo):
        u = u + gcol * w1g[kk:kk + 1, :]
    xx = jax.nn.silu(u)
    x2h = jnp.dot(xx, w2_ref[...], preferred_element_type=F32) + b2_ref[...]
    m = jnp.mean(x2h, axis=1, keepdims=True)
    v = jnp.mean((x2h - m) ** 2, axis=1, keepdims=True)
    ef = (x2h - m) / jnp.sqrt(v + 1e-5) * g_ref[...] + lb_ref[...]
    wh = jax.nn.silu(jnp.dot(ef, wc1_ref[...], preferred_element_type=F32)
                     + bc1_ref[...])
    w = jnp.sum(wh * wc2_ref[...], axis=1, keepdims=True)
    trans = jnp.concatenate([r0 * w, r1 * w, r2 * w], axis=1)
    pad = jnp.zeros((ef.shape[0], TW - HID - 3), F32)
    out_ref[...] = jnp.concatenate([ef, trans, pad], axis=1)


def _edge_call(trow, tcol, eap, ew):
    (w1a, w1b, w1g, w1e, b1, w2, b2, g, lb, wc1, bc1, wc2) = ew
    grid = (EPAD // BE,)
    blk = lambda w: pl.BlockSpec((BE, w), lambda i: (i, 0))
    return pl.pallas_call(
        _edge_body,
        grid=grid,
        in_specs=[blk(TW), blk(TW), blk(8),
                  _full(w1a.shape), _full(w1b.shape), _full(w1g.shape),
                  _full(w1e.shape), _full(b1.shape), _full(w2.shape),
                  _full(b2.shape), _full(g.shape), _full(lb.shape),
                  _full(wc1.shape), _full(bc1.shape), _full(wc2.shape)],
        out_specs=blk(TW),
        out_shape=jax.ShapeDtypeStruct((EPAD, TW), F32),
    )(trow, tcol, eap, w1a, w1b, w1g, w1e, b1, w2, b2, g, lb, wc1, bc1, wc2)


def _node_body(t_ref, p0_ref, p1_ref, wn1a_ref, wn1b_ref, bn1_ref, wn2_ref,
               bn2_ref, out_ref):
    t = t_ref[...]
    p0 = p0_ref[...]
    p1 = p1_ref[...]
    h = t[:, :HID]
    agg = p0[:, :HID] + p1[:, :HID]
    u = (jnp.dot(h, wn1a_ref[...], preferred_element_type=F32)
         + jnp.dot(agg, wn1b_ref[...], preferred_element_type=F32)
         + bn1_ref[...])
    hn = h + jnp.dot(jax.nn.silu(u), wn2_ref[...],
                     preferred_element_type=F32) + bn2_ref[...]
    xn = t[:, HID:HID + 3] + p0[:, HID:HID + 3] + p1[:, HID:HID + 3]
    pad = jnp.zeros((t.shape[0], TW - HID - 3), F32)
    out_ref[...] = jnp.concatenate([hn, xn, pad], axis=1)


def _node_call(t, p0, p1, nw):
    wn1a, wn1b, bn1, wn2, bn2 = nw
    grid = (NPAD // BN,)
    blk = pl.BlockSpec((BN, TW), lambda i: (i, 0))
    return pl.pallas_call(
        _node_body,
        grid=grid,
        in_specs=[blk, blk, blk, _full(wn1a.shape), _full(wn1b.shape),
                  _full(bn1.shape), _full(wn2.shape), _full(bn2.shape)],
        out_specs=blk,
        out_shape=jax.ShapeDtypeStruct((NPAD, TW), F32),
    )(t, p0, p1, wn1a, wn1b, bn1, wn2, bn2)


def _embin_body(h_ref, x_ref, w_ref, b_ref, out_ref):
    h1 = jnp.dot(h_ref[...], w_ref[...], preferred_element_type=F32) + b_ref[...]
    out_ref[...] = jnp.concatenate([h1, x_ref[...]], axis=1)


def _embin_call(h0p, x16, w, b):
    grid = (NPAD // BN,)
    return pl.pallas_call(
        _embin_body,
        grid=grid,
        in_specs=[pl.BlockSpec((BN, 128), lambda i: (i, 0)),
                  pl.BlockSpec((BN, 16), lambda i: (i, 0)),
                  _full(w.shape), _full(b.shape)],
        out_specs=pl.BlockSpec((BN, TW), lambda i: (i, 0)),
        out_shape=jax.ShapeDtypeStruct((NPAD, TW), F32),
    )(h0p, x16, w, b)


def _embout_body(ts_ref, tt_ref, wt_ref, b_ref, out_ref):
    wt = wt_ref[...]
    b = b_ref[...]
    hs = ts_ref[...][:, :HID]
    ht = tt_ref[...][:, :HID]
    dn = (((1,), (1,)), ((), ()))
    a = lax.dot_general(wt, hs, dn, preferred_element_type=F32) + b
    c = lax.dot_general(wt, ht, dn, preferred_element_type=F32) + b
    out_ref[...] = jnp.concatenate([a, c], axis=0)


def _embout_call(ts, tt, wt, b):
    grid = (NPAD // BN,)
    blk = pl.BlockSpec((BN, TW), lambda i: (i, 0))
    return pl.pallas_call(
        _embout_body,
        grid=grid,
        in_specs=[blk, blk, _full(wt.shape), _full(b.shape)],
        out_specs=pl.BlockSpec((128, BN), lambda i: (0, i)),
        out_shape=jax.ShapeDtypeStruct((128, NPAD), F32),
    )(ts, tt, wt, b)


def _compress_body(h2_ref, w1_ref, b1_ref, w2_ref, b2_ref, out_ref, acc_ref):
    kstep = pl.program_id(0)

    @pl.when(kstep == 0)
    def _():
        acc_ref[...] = jnp.zeros_like(acc_ref)

    col = kstep * BN + lax.broadcasted_iota(jnp.int32, (128, BN), 1)
    h2b = jnp.where(col < NN, h2_ref[...], 0.0)
    acc_ref[...] += jnp.dot(h2b, w1_ref[...], preferred_element_type=F32)

    @pl.when(kstep == pl.num_programs(0) - 1)
    def _():
        r = jnp.maximum(acc_ref[...] + b1_ref[...], 0.0)
        out_ref[...] = jnp.dot(r, w2_ref[...],
                               preferred_element_type=F32) + b2_ref[...]


def _compress_call(h2, w1, b1, w2, b2):
    nk = NPAD // BN
    grid = (nk,)
    return pl.pallas_call(
        _compress_body,
        grid=grid,
        in_specs=[pl.BlockSpec((128, BN), lambda i: (0, i)),
                  pl.BlockSpec((BN, 2500), lambda i: (i, 0)),
                  _full(b1.shape), _full(w2.shape), _full(b2.shape)],
        out_specs=_full((128, 128)),
        out_shape=jax.ShapeDtypeStruct((128, 128), F32),
        scratch_shapes=[pltpu.VMEM((128, 2500), F32)],
    )(h2, w1, b1, w2, b2)


def _pose_body(z_ref, w0_ref, b0_ref, w1_ref, b1_ref, w2_ref, b2_ref,
               w3_ref, b3_ref, out_ref):
    a = jnp.maximum(jnp.dot(z_ref[...], w0_ref[...],
                            preferred_element_type=F32) + b0_ref[...], 0.0)
    a = jnp.maximum(jnp.dot(a, w1_ref[...],
                            preferred_element_type=F32) + b1_ref[...], 0.0)
    a = jnp.maximum(jnp.dot(a, w2_ref[...],
                            preferred_element_type=F32) + b2_ref[...], 0.0)
    p = jnp.dot(a, w3_ref[...], preferred_element_type=F32) + b3_ref[...]
    lanes = lax.broadcasted_iota(jnp.int32, (1, 8), 1)
    qmask = lanes < 4
    qn = jnp.sqrt(jnp.sum(jnp.where(qmask, p, 0.0) ** 2))
    out_ref[...] = jnp.where(qmask, p / (qn + 1e-8), p)


def _pose_call(z, pw):
    w0, b0, w1, b1, w2, b2, w3, b3 = pw
    return pl.pallas_call(
        _pose_body,
        grid=(1,),
        in_specs=[_full(z.shape), _full(w0.shape), _full(b0.shape),
                  _full(w1.shape), _full(b1.shape), _full(w2.shape),
                  _full(b2.shape), _full(w3.shape), _full(b3.shape)],
        out_specs=_full((1, 8)),
        out_shape=jax.ShapeDtypeStruct((1, 8), F32),
    )(z, w0, b0, w1, b1, w2, b2, w3, b3)


def _prep_layer(lp):
    w1 = jnp.concatenate([m[0]["W"] for m in lp["edge_mlps"]], axis=1)
    b1 = jnp.concatenate([m[0]["b"] for m in lp["edge_mlps"]])[None, :]
    w1a = w1[:HID]
    w1b = w1[HID:2 * HID]
    w1g = jnp.zeros((16, HID), F32).at[:12].set(w1[128:140])
    w1e = jnp.zeros((8, HID), F32).at[:4].set(w1[140:144])
    w2 = jnp.zeros((HID, HID), F32)
    for kk, m in enumerate(lp["edge_mlps"]):
        w2 = w2.at[16 * kk:16 * kk + 16, 16 * kk:16 * kk + 16].set(m[1]["W"])
    b2 = jnp.concatenate([m[1]["b"] for m in lp["edge_mlps"]])[None, :]
    g = lp["ln"]["g"][None, :]
    lb = lp["ln"]["b"][None, :]
    wc1 = lp["coord_mlp"][0]["W"]
    bc1 = lp["coord_mlp"][0]["b"][None, :]
    wc2 = lp["coord_mlp"][1]["W"].T
    ew = (w1a, w1b, w1g, w1e, b1, w2, b2, g, lb, wc1, bc1, wc2)
    wn1 = lp["node_mlp"][0]["W"]
    nw = (wn1[:HID], wn1[HID:], lp["node_mlp"][0]["b"][None, :],
          lp["node_mlp"][1]["W"], lp["node_mlp"][1]["b"][None, :])
    return ew, nw


def _prep_edges(edges, edge_attr):
    row = edges[0].astype(jnp.int32)
    col = edges[1].astype(jnp.int32)
    npad = EPAD - EE
    grow = jnp.concatenate([row, jnp.zeros((npad,), jnp.int32)])
    gcol = jnp.concatenate([col, jnp.zeros((npad,), jnp.int32)])
    srow = jnp.concatenate([row, jnp.full((npad,), NN, jnp.int32)])
    eap = jnp.zeros((EPAD, 8), F32).at[:EE, :4].set(edge_attr)
    return (grow.reshape(-1, CH), gcol.reshape(-1, CH),
            srow.reshape(-1, CH), eap)


def _forward2(src_in, tgt_in, p, layer_ws, zero128):
    ts = [None, None]
    es = []
    for g, (h0, x0, edges, edge_attr) in enumerate((src_in, tgt_in)):
        grow, gcol, srow, eap = _prep_edges(edges, edge_attr)
        es.append((grow, gcol, srow, eap))
        h0p = jnp.pad(h0, ((0, NPAD - NN), (0, 0)))
        x16 = jnp.pad(x0, ((0, NPAD - NN), (0, 13)))
        ts[g] = _embin_call(h0p, x16, p["emb_in"]["W"],
                            p["emb_in"]["b"][None, :])
    for ew, nw in layer_ws:
        gath = [_sc_gather(ts[g], es[g][0], es[g][1]) for g in range(2)]
        oute = [_edge_call(gath[g][0], gath[g][1], es[g][3], ew)
                for g in range(2)]
        part = [_sc_scatter(oute[g], es[g][2], zero128) for g in range(2)]
        ts = [_node_call(ts[g], part[g][0], part[g][1], nw) for g in range(2)]
    return ts[0], ts[1]


def kernel(h_src, x_src, edges_src, edge_attr_src, h_tgt, x_tgt, edges_tgt,
           edge_attr_tgt, corr, labels, params):
    p = params
    layer_ws = [_prep_layer(lp) for lp in p["layers"]]
    zero128 = jnp.zeros((CH, TW), F32)

    ts, tt = _forward2((h_src, x_src, edges_src, edge_attr_src),
                       (h_tgt, x_tgt, edges_tgt, edge_attr_tgt),
                       p, layer_ws, zero128)

    wt = p["emb_out"]["W"].T
    bout = p["emb_out"]["b"][:, None]
    h2 = _embout_call(ts, tt, wt, bout)

    cz = _compress_call(h2, p["compress"][0]["W"],
                        p["compress"][0]["b"][None, :],
                        p["compress"][1]["W"], p["compress"][1]["b"][None, :])
    z = cz.reshape(1, 128 * 128)

    pw = (p["pose"][0]["W"], p["pose"][0]["b"][None, :],
          p["pose"][1]["W"], p["pose"][1]["b"][None, :],
          p["pose"][2]["W"], p["pose"][2]["b"][None, :],
          jnp.zeros((64, 8), F32).at[:, :7].set(p["pose"][3]["W"]),
          jnp.zeros((1, 8), F32).at[:, :7].set(p["pose"][3]["b"][None, :]))
    pose = _pose_call(z, pw)
    return pose.reshape(8)[:7]


# speedup vs baseline: 2.2252x; 2.2252x over previous
import functools

import jax
import jax.numpy as jnp
from jax import lax
from jax.experimental import pallas as pl
from jax.experimental.pallas import tpu as pltpu
from jax.experimental.pallas import tpu_sc as plsc

NN = 10000
NPAD = 10240
EE = 160000
EPAD = 163840
TW = 80
CH = 128
NWORK = 32
CPW = EPAD // (NWORK * CH)
HID = 64
BE = 2048
BN = 1024
F32 = jnp.float32


@functools.lru_cache(maxsize=None)
def _gather_kernel():
    mesh = plsc.VectorSubcoreMesh(core_axis_name="c", subcore_axis_name="s")

    @functools.partial(
        pl.kernel, mesh=mesh,
        compiler_params=pltpu.CompilerParams(use_tc_tiling_on_sc=False),
        out_type=(jax.ShapeDtypeStruct((EPAD, TW), F32),
                  jax.ShapeDtypeStruct((EPAD, TW), F32)),
        scratch_types=[pltpu.VMEM((CPW, CH), jnp.int32),
                       pltpu.VMEM((CPW, CH), jnp.int32),
                       pltpu.VMEM((2, CH, TW), F32),
                       pltpu.VMEM((2, CH, TW), F32),
                       [pltpu.SemaphoreType.DMA] * 2,
                       [pltpu.SemaphoreType.DMA] * 2,
                       [pltpu.SemaphoreType.DMA] * 2,
                       [pltpu.SemaphoreType.DMA] * 2])
    def k(t_hbm, row_hbm, col_hbm, orow_hbm, ocol_hbm,
          idxr_v, idxc_v, rbuf_v, cbuf_v, sgr, sgc, sor, soc):
        wid = lax.axis_index("s") * 2 + lax.axis_index("c")
        base = wid * CPW
        pltpu.sync_copy(row_hbm.at[pl.ds(base, CPW)], idxr_v)
        pltpu.sync_copy(col_hbm.at[pl.ds(base, CPW)], idxc_v)

        def start_g(j, b):
            pltpu.async_copy(t_hbm.at[idxr_v.at[j]], rbuf_v.at[b], sgr[b])
            pltpu.async_copy(t_hbm.at[idxc_v.at[j]], cbuf_v.at[b], sgc[b])

        def wait_g(b):
            dummy = orow_hbm.at[pl.ds(0, CH)]
            pltpu.make_async_copy(dummy, rbuf_v.at[b], sgr[b]).wait()
            pltpu.make_async_copy(dummy, cbuf_v.at[b], sgc[b]).wait()

        for b in range(2):
            start_g(b, b)

        def body(i, carry):
            g = i * 2
            for b in range(2):
                j = g + b
                eb = (base + j) * CH
                wait_g(b)
                pltpu.async_copy(rbuf_v.at[b], orow_hbm.at[pl.ds(eb, CH)],
                                 sor[b])
                pltpu.async_copy(cbuf_v.at[b], ocol_hbm.at[pl.ds(eb, CH)],
                                 soc[b])
                pltpu.make_async_copy(rbuf_v.at[b],
                                      orow_hbm.at[pl.ds(eb, CH)],
                                      sor[b]).wait()
                pltpu.make_async_copy(cbuf_v.at[b],
                                      ocol_hbm.at[pl.ds(eb, CH)],
                                      soc[b]).wait()

                @pl.when(j + 2 < CPW)
                def _():
                    start_g(j + 2, b)

            return carry

        lax.fori_loop(0, CPW // 2, body, 0)

    return k


@functools.lru_cache(maxsize=None)
def _scatter_kernel():
    mesh = plsc.VectorSubcoreMesh(core_axis_name="c", subcore_axis_name="s")
    zchunks = NPAD // CH // 16

    @functools.partial(
        pl.kernel, mesh=mesh,
        compiler_params=pltpu.CompilerParams(use_tc_tiling_on_sc=False),
        out_type=jax.ShapeDtypeStruct((2, NPAD, TW), F32),
        scratch_types=[pltpu.VMEM((CPW, CH), jnp.int32),
                       pltpu.VMEM((CH, TW), F32),
                       pltpu.VMEM_SHARED((NPAD, TW), F32)])
    def k(vals_hbm, row_hbm, zero_hbm, p_hbm, idx_v, buf_v, acc_s):
        cid = lax.axis_index("c")
        sid = lax.axis_index("s")
        wid = sid * 2 + cid
        pltpu.sync_copy(zero_hbm, buf_v)

        def zbody(j, carry):
            pltpu.sync_copy(buf_v, acc_s.at[pl.ds((sid * zchunks + j) * CH, CH)])
            return carry

        lax.fori_loop(0, zchunks, zbody, 0)
        plsc.subcore_barrier()

        base = wid * CPW
        pltpu.sync_copy(row_hbm.at[pl.ds(base, CPW)], idx_v)

        def body(j, carry):
            eb = (base + j) * CH
            pltpu.sync_copy(vals_hbm.at[pl.ds(eb, CH)], buf_v)
            pltpu.sync_copy(buf_v, acc_s.at[idx_v.at[j]], add=True)
            return carry

        lax.fori_loop(0, CPW, body, 0)
        plsc.subcore_barrier()

        def obody(j, carry):
            off = (sid * zchunks + j) * CH
            pltpu.sync_copy(acc_s.at[pl.ds(off, CH)], buf_v)
            pltpu.sync_copy(buf_v, p_hbm.at[cid, pl.ds(off, CH)])
            return carry

        lax.fori_loop(0, zchunks, obody, 0)

    return k


def _sc_gather(t, row2d, col2d):
    return _gather_kernel()(t, row2d, col2d)


def _sc_scatter(vals, row2d, zero128):
    return _scatter_kernel()(vals, row2d, zero128)


def _full(shape):
    nd = len(shape)
    return pl.BlockSpec(shape, lambda i: (0,) * nd)


def _edge_body(tr_ref, tc_ref, ea_ref, w1a_ref, w1b_ref, w1g_ref, w1e_ref,
               b1_ref, w2_ref, b2_ref, g_ref, lb_ref, wc1_ref, bc1_ref,
               wc2_ref, out_ref):
    tr = tr_ref[...]
    tc = tc_ref[...]
    hr = tr[:, :HID]
    hc = tc[:, :HID]
    eye8 = jnp.eye(8, dtype=F32)
    dn_bt = (((1,), (1,)), ((), ()))
    crt = lax.dot_general(eye8, tr[:, 64:72], dn_bt,
                          preferred_element_type=F32)
    cct = lax.dot_general(eye8, tc[:, 64:72], dn_bt,
                          preferred_element_type=F32)
    x = crt[0:3]
    y = cct[0:3]
    r = x - y
    rad = jnp.sum(r * r, axis=0, keepdims=True)
    dist = jnp.sqrt(rad)
    dot = jnp.sum(x * y, axis=0, keepdims=True)
    inv_r = 1.0 / (dist + 1e-8)
    a = r * inv_r
    x0, x1, x2 = crt[0:1], crt[1:2], crt[2:3]
    y0, y1, y2 = cct[0:1], cct[1:2], cct[2:3]
    c0 = x1 * y2 - x2 * y1
    c1 = x2 * y0 - x0 * y2
    c2 = x0 * y1 - x1 * y0
    cn = jnp.sqrt(c0 * c0 + c1 * c1 + c2 * c2)
    inv_c = 1.0 / (cn + 1e-8)
    b0, b1v, b2v = c0 * inv_c, c1 * inv_c, c2 * inv_c
    a0, a1, a2 = a[0:1], a[1:2], a[2:3]
    e0 = a1 * b2v - a2 * b1v
    e1 = a2 * b0 - a0 * b2v
    e2 = a0 * b1v - a1 * b0
    na = jnp.sqrt(a0 * a0 + a1 * a1 + a2 * a2)
    nb = jnp.sqrt(b0 * b0 + b1v * b1v + b2v * b2v)
    nc = jnp.sqrt(e0 * e0 + e1 * e1 + e2 * e2)
    thr = 1e-6
    mask = (na < thr) | (nb < thr) | (nc < thr)
    so3 = [(a0, 1.0), (b0, 0.0), (e0, 0.0),
           (a1, 0.0), (b1v, 1.0), (e1, 0.0),
           (a2, 0.0), (b2v, 0.0), (e2, 1.0)]
    geo = [rad, dist, dot] + [jnp.where(mask, ident, v) for v, ident in so3]
    gt = jnp.concatenate(geo + [jnp.zeros((4, tr.shape[0]), F32)],
                         axis=0)

    dn_gt = (((0,), (0,)), ((), ()))
    u = (jnp.dot(hr, w1a_ref[...], preferred_element_type=F32)
         + jnp.dot(hc, w1b_ref[...], preferred_element_type=F32)
         + jnp.dot(ea_ref[...], w1e_ref[...], preferred_element_type=F32)
         + lax.dot_general(gt, w1g_ref[...], dn_gt, preferred_element_type=F32)
         + b1_ref[...])
    xx = jax.nn.silu(u)
    x2h = jnp.dot(xx, w2_ref[...], preferred_element_type=F32) + b2_ref[...]
    m = jnp.mean(x2h, axis=1, keepdims=True)
    v = jnp.mean((x2h - m) ** 2, axis=1, keepdims=True)
    ef = (x2h - m) / jnp.sqrt(v + 1e-5) * g_ref[...] + lb_ref[...]
    wh = jax.nn.silu(jnp.dot(ef, wc1_ref[...], preferred_element_type=F32)
                     + bc1_ref[...])
    wt = lax.dot_general(wc2_ref[...], wh, dn_bt,
                         preferred_element_type=F32)
    t8 = jnp.concatenate([r * wt, jnp.zeros((5, tr.shape[0]), F32)], axis=0)
    trans8 = lax.dot_general(t8, eye8, dn_gt,
                             preferred_element_type=F32)
    pad = jnp.zeros((tr.shape[0], TW - HID - 8), F32)
    out_ref[...] = jnp.concatenate([ef, trans8, pad], axis=1)


def _edge_call(trow, tcol, eap, ew):
    (w1a, w1b, w1g, w1e, b1, w2, b2, g, lb, wc1, bc1, wc2) = ew
    grid = (EPAD // BE,)
    blk = lambda w: pl.BlockSpec((BE, w), lambda i: (i, 0))
    return pl.pallas_call(
        _edge_body,
        grid=grid,
        in_specs=[blk(TW), blk(TW), blk(8),
                  _full(w1a.shape), _full(w1b.shape), _full(w1g.shape),
                  _full(w1e.shape), _full(b1.shape), _full(w2.shape),
                  _full(b2.shape), _full(g.shape), _full(lb.shape),
                  _full(wc1.shape), _full(bc1.shape), _full(wc2.shape)],
        out_specs=blk(TW),
        out_shape=jax.ShapeDtypeStruct((EPAD, TW), F32),
    )(trow, tcol, eap, w1a, w1b, w1g, w1e, b1, w2, b2, g, lb, wc1, bc1, wc2)


def _node_body(t_ref, p0_ref, p1_ref, wn1a_ref, wn1b_ref, bn1_ref, wn2_ref,
               bn2_ref, out_ref):
    t = t_ref[...]
    p0 = p0_ref[...]
    p1 = p1_ref[...]
    h = t[:, :HID]
    agg = p0[:, :HID] + p1[:, :HID]
    u = (jnp.dot(h, wn1a_ref[...], preferred_element_type=F32)
         + jnp.dot(agg, wn1b_ref[...], preferred_element_type=F32)
         + bn1_ref[...])
    hn = h + jnp.dot(jax.nn.silu(u), wn2_ref[...],
                     preferred_element_type=F32) + bn2_ref[...]
    xn = t[:, HID:HID + 3] + p0[:, HID:HID + 3] + p1[:, HID:HID + 3]
    pad = jnp.zeros((t.shape[0], TW - HID - 3), F32)
    out_ref[...] = jnp.concatenate([hn, xn, pad], axis=1)


def _node_call(t, p0, p1, nw):
    wn1a, wn1b, bn1, wn2, bn2 = nw
    grid = (NPAD // BN,)
    blk = pl.BlockSpec((BN, TW), lambda i: (i, 0))
    return pl.pallas_call(
        _node_body,
        grid=grid,
        in_specs=[blk, blk, blk, _full(wn1a.shape), _full(wn1b.shape),
                  _full(bn1.shape), _full(wn2.shape), _full(bn2.shape)],
        out_specs=blk,
        out_shape=jax.ShapeDtypeStruct((NPAD, TW), F32),
    )(t, p0, p1, wn1a, wn1b, bn1, wn2, bn2)


def _embin_body(h_ref, x_ref, w_ref, b_ref, out_ref):
    h1 = jnp.dot(h_ref[...], w_ref[...], preferred_element_type=F32) + b_ref[...]
    out_ref[...] = jnp.concatenate([h1, x_ref[...]], axis=1)


def _embin_call(h0p, x16, w, b):
    grid = (NPAD // BN,)
    return pl.pallas_call(
        _embin_body,
        grid=grid,
        in_specs=[pl.BlockSpec((BN, 128), lambda i: (i, 0)),
                  pl.BlockSpec((BN, 16), lambda i: (i, 0)),
                  _full(w.shape), _full(b.shape)],
        out_specs=pl.BlockSpec((BN, TW), lambda i: (i, 0)),
        out_shape=jax.ShapeDtypeStruct((NPAD, TW), F32),
    )(h0p, x16, w, b)


def _embout_body(ts_ref, tt_ref, wt_ref, b_ref, out_ref):
    wt = wt_ref[...]
    b = b_ref[...]
    hs = ts_ref[...][:, :HID]
    ht = tt_ref[...][:, :HID]
    dn = (((1,), (1,)), ((), ()))
    a = lax.dot_general(wt, hs, dn, preferred_element_type=F32) + b
    c = lax.dot_general(wt, ht, dn, preferred_element_type=F32) + b
    out_ref[...] = jnp.concatenate([a, c], axis=0)


def _embout_call(ts, tt, wt, b):
    grid = (NPAD // BN,)
    blk = pl.BlockSpec((BN, TW), lambda i: (i, 0))
    return pl.pallas_call(
        _embout_body,
        grid=grid,
        in_specs=[blk, blk, _full(wt.shape), _full(b.shape)],
        out_specs=pl.BlockSpec((128, BN), lambda i: (0, i)),
        out_shape=jax.ShapeDtypeStruct((128, NPAD), F32),
    )(ts, tt, wt, b)


def _compress_body(h2_ref, w1_ref, b1_ref, w2_ref, b2_ref, out_ref, acc_ref):
    kstep = pl.program_id(0)

    @pl.when(kstep == 0)
    def _():
        acc_ref[...] = jnp.zeros_like(acc_ref)

    col = kstep * BN + lax.broadcasted_iota(jnp.int32, (128, BN), 1)
    h2b = jnp.where(col < NN, h2_ref[...], 0.0)
    acc_ref[...] += jnp.dot(h2b, w1_ref[...], preferred_element_type=F32)

    @pl.when(kstep == pl.num_programs(0) - 1)
    def _():
        r = jnp.maximum(acc_ref[...] + b1_ref[...], 0.0)
        out_ref[...] = jnp.dot(r, w2_ref[...],
                               preferred_element_type=F32) + b2_ref[...]


def _compress_call(h2, w1, b1, w2, b2):
    nk = NPAD // BN
    grid = (nk,)
    return pl.pallas_call(
        _compress_body,
        grid=grid,
        in_specs=[pl.BlockSpec((128, BN), lambda i: (0, i)),
                  pl.BlockSpec((BN, 2500), lambda i: (i, 0)),
                  _full(b1.shape), _full(w2.shape), _full(b2.shape)],
        out_specs=_full((128, 128)),
        out_shape=jax.ShapeDtypeStruct((128, 128), F32),
        scratch_shapes=[pltpu.VMEM((128, 2500), F32)],
    )(h2, w1, b1, w2, b2)


def _pose_body(z_ref, w0_ref, b0_ref, w1_ref, b1_ref, w2_ref, b2_ref,
               w3_ref, b3_ref, out_ref):
    a = jnp.maximum(jnp.dot(z_ref[...], w0_ref[...],
                            preferred_element_type=F32) + b0_ref[...], 0.0)
    a = jnp.maximum(jnp.dot(a, w1_ref[...],
                            preferred_element_type=F32) + b1_ref[...], 0.0)
    a = jnp.maximum(jnp.dot(a, w2_ref[...],
                            preferred_element_type=F32) + b2_ref[...], 0.0)
    p = jnp.dot(a, w3_ref[...], preferred_element_type=F32) + b3_ref[...]
    lanes = lax.broadcasted_iota(jnp.int32, (1, 8), 1)
    qmask = lanes < 4
    qn = jnp.sqrt(jnp.sum(jnp.where(qmask, p, 0.0) ** 2))
    out_ref[...] = jnp.where(qmask, p / (qn + 1e-8), p)


def _pose_call(z, pw):
    w0, b0, w1, b1, w2, b2, w3, b3 = pw
    return pl.pallas_call(
        _pose_body,
        grid=(1,),
        in_specs=[_full(z.shape), _full(w0.shape), _full(b0.shape),
                  _full(w1.shape), _full(b1.shape), _full(w2.shape),
                  _full(b2.shape), _full(w3.shape), _full(b3.shape)],
        out_specs=_full((1, 8)),
        out_shape=jax.ShapeDtypeStruct((1, 8), F32),
    )(z, w0, b0, w1, b1, w2, b2, w3, b3)


def _prep_layer(lp):
    w1 = jnp.concatenate([m[0]["W"] for m in lp["edge_mlps"]], axis=1)
    b1 = jnp.concatenate([m[0]["b"] for m in lp["edge_mlps"]])[None, :]
    w1a = w1[:HID]
    w1b = w1[HID:2 * HID]
    w1g = jnp.zeros((16, HID), F32).at[:12].set(w1[128:140])
    w1e = jnp.zeros((8, HID), F32).at[:4].set(w1[140:144])
    w2 = jnp.zeros((HID, HID), F32)
    for kk, m in enumerate(lp["edge_mlps"]):
        w2 = w2.at[16 * kk:16 * kk + 16, 16 * kk:16 * kk + 16].set(m[1]["W"])
    b2 = jnp.concatenate([m[1]["b"] for m in lp["edge_mlps"]])[None, :]
    g = lp["ln"]["g"][None, :]
    lb = lp["ln"]["b"][None, :]
    wc1 = lp["coord_mlp"][0]["W"]
    bc1 = lp["coord_mlp"][0]["b"][None, :]
    wc2 = lp["coord_mlp"][1]["W"].T
    ew = (w1a, w1b, w1g, w1e, b1, w2, b2, g, lb, wc1, bc1, wc2)
    wn1 = lp["node_mlp"][0]["W"]
    nw = (wn1[:HID], wn1[HID:], lp["node_mlp"][0]["b"][None, :],
          lp["node_mlp"][1]["W"], lp["node_mlp"][1]["b"][None, :])
    return ew, nw


def _prep_edges(edges, edge_attr):
    row = edges[0].astype(jnp.int32)
    col = edges[1].astype(jnp.int32)
    npad = EPAD - EE
    grow = jnp.concatenate([row, jnp.zeros((npad,), jnp.int32)])
    gcol = jnp.concatenate([col, jnp.zeros((npad,), jnp.int32)])
    srow = jnp.concatenate([row, jnp.full((npad,), NN, jnp.int32)])
    eap = jnp.zeros((EPAD, 8), F32).at[:EE, :4].set(edge_attr)
    return (grow.reshape(-1, CH), gcol.reshape(-1, CH),
            srow.reshape(-1, CH), eap)


def _forward2(src_in, tgt_in, p, layer_ws, zero128):
    ts = [None, None]
    es = []
    for g, (h0, x0, edges, edge_attr) in enumerate((src_in, tgt_in)):
        grow, gcol, srow, eap = _prep_edges(edges, edge_attr)
        es.append((grow, gcol, srow, eap))
        h0p = jnp.pad(h0, ((0, NPAD - NN), (0, 0)))
        x16 = jnp.pad(x0, ((0, NPAD - NN), (0, 13)))
        ts[g] = _embin_call(h0p, x16, p["emb_in"]["W"],
                            p["emb_in"]["b"][None, :])
    for ew, nw in layer_ws:
        gath = [_sc_gather(ts[g], es[g][0], es[g][1]) for g in range(2)]
        oute = [_edge_call(gath[g][0], gath[g][1], es[g][3], ew)
                for g in range(2)]
        part = [_sc_scatter(oute[g], es[g][2], zero128) for g in range(2)]
        ts = [_node_call(ts[g], part[g][0], part[g][1], nw) for g in range(2)]
    return ts[0], ts[1]


def kernel(h_src, x_src, edges_src, edge_attr_src, h_tgt, x_tgt, edges_tgt,
           edge_attr_tgt, corr, labels, params):
    p = params
    layer_ws = [_prep_layer(lp) for lp in p["layers"]]
    zero128 = jnp.zeros((CH, TW), F32)

    ts, tt = _forward2((h_src, x_src, edges_src, edge_attr_src),
                       (h_tgt, x_tgt, edges_tgt, edge_attr_tgt),
                       p, layer_ws, zero128)

    wt = p["emb_out"]["W"].T
    bout = p["emb_out"]["b"][:, None]
    h2 = _embout_call(ts, tt, wt, bout)

    cz = _compress_call(h2, p["compress"][0]["W"],
                        p["compress"][0]["b"][None, :],
                        p["compress"][1]["W"], p["compress"][1]["b"][None, :])
    z = cz.reshape(1, 128 * 128)

    pw = (p["pose"][0]["W"], p["pose"][0]["b"][None, :],
          p["pose"][1]["W"], p["pose"][1]["b"][None, :],
          p["pose"][2]["W"], p["pose"][2]["b"][None, :],
          jnp.zeros((64, 8), F32).at[:, :7].set(p["pose"][3]["W"]),
          jnp.zeros((1, 8), F32).at[:, :7].set(p["pose"][3]["b"][None, :]))
    pose = _pose_call(z, pw)
    return pose.reshape(8)[:7]
